# baseline (speedup 1.0000x reference)
import jax
import jax.numpy as jnp
from jax.experimental import pallas as pl
from jax.experimental.pallas import tpu as pltpu

_T = 12
_B = 8
_N = 512
_D = 64
_P = 3
_O = 64

_RHS_T = (((1,), (1,)), ((), ()))
_BB = 2


def _gg_kernel(x_ref, s_ref, w1_ref, b1_ref, w2_ref, b2_ref, out_ref,
               s16_scr, w1_scr, w2_scr, b1_scr, b2_scr):
    @pl.when(pl.program_id(0) == 0)
    def _init():
        s16_scr[...] = s_ref[...].astype(jnp.bfloat16)
        w1_scr[...] = w1_ref[...].astype(jnp.bfloat16)
        w2_scr[...] = w2_ref[...].astype(jnp.bfloat16)
        b1_scr[...] = jnp.transpose(b1_ref[...], (1, 0))
        b2_scr[...] = jnp.transpose(b2_ref[...], (1, 0))

    w1 = w1_scr[...]
    w2 = w2_scr[...]
    b1 = b1_scr[...]
    b2 = b2_scr[...]
    for j in range(_BB):
        xall = x_ref[:, j].reshape(_T * _D, _N)
        yall = jax.lax.dot_general(xall, s16_scr[...], _RHS_T,
                                   preferred_element_type=jnp.float32)
        yall = yall.astype(jnp.bfloat16)
        for i in range(_T):
            lo = i * _D
            hi = min(lo + _P * _D, _T * _D)
            k = hi - lo
            ywin = yall[lo:hi]
            xwin = xall[lo:hi]
            lin = jnp.dot(w1[:, :k], ywin, preferred_element_type=jnp.float32) + b1
            inp2 = jnp.dot(w2[:, :k], xwin, preferred_element_type=jnp.float32) + b2
            xh = lin[:_O]
            gate = lin[_O:].astype(jnp.bfloat16)
            sg = jax.nn.sigmoid(gate).astype(jnp.float32)
            out_ref[i, j] = (xh + inp2) * sg


def kernel(x, supports, W1, b1, W2, b2):
    t, b, n, d = x.shape
    xp = jnp.transpose(x, (0, 1, 3, 2)).astype(jnp.bfloat16)
    b1r = b1.reshape(1, -1)
    b2r = b2.reshape(1, -1)

    out = pl.pallas_call(
        _gg_kernel,
        grid=(b // _BB,),
        in_specs=[
            pl.BlockSpec((t, _BB, d, n), lambda i: (0, i, 0, 0)),
            pl.BlockSpec((n, n), lambda i: (0, 0)),
            pl.BlockSpec((2 * _O, _P * d), lambda i: (0, 0)),
            pl.BlockSpec((1, 2 * _O), lambda i: (0, 0)),
            pl.BlockSpec((_O, _P * d), lambda i: (0, 0)),
            pl.BlockSpec((1, _O), lambda i: (0, 0)),
        ],
        out_specs=pl.BlockSpec((t, _BB, _O, n), lambda i: (0, i, 0, 0)),
        out_shape=jax.ShapeDtypeStruct((t, b, _O, n), jnp.float32),
        scratch_shapes=[
            pltpu.VMEM((n, n), jnp.bfloat16),
            pltpu.VMEM((2 * _O, _P * d), jnp.bfloat16),
            pltpu.VMEM((_O, _P * d), jnp.bfloat16),
            pltpu.VMEM((2 * _O, 1), jnp.float32),
            pltpu.VMEM((_O, 1), jnp.float32),
        ],
    )(xp, supports, W1, b1r, W2, b2r)
    return jnp.transpose(out, (0, 1, 3, 2))


# speedup vs baseline: 1.4194x; 1.4194x over previous
import jax
import jax.numpy as jnp
from jax.experimental import pallas as pl
from jax.experimental.pallas import tpu as pltpu

_T = 12
_B = 8
_N = 512
_D = 64
_P = 3
_O = 64

_RHS_T = (((1,), (1,)), ((), ()))
_BB = 2


def _gg_kernel(x_ref, s_ref, w1_ref, b1_ref, w2_ref, b2_ref, out_ref,
               s16_scr, w1_scr, w2_scr, b1_scr, b2_scr):
    @pl.when(pl.program_id(0) == 0)
    def _init():
        s16_scr[...] = s_ref[...].astype(jnp.bfloat16)
        w1_scr[...] = w1_ref[...].astype(jnp.bfloat16)
        w2_scr[...] = w2_ref[...].astype(jnp.bfloat16)
        b1_scr[...] = jnp.transpose(b1_ref[...], (1, 0))
        b2_scr[...] = jnp.transpose(b2_ref[...], (1, 0))

    w1 = w1_scr[...]
    w2 = w2_scr[...]
    b1 = b1_scr[...]
    b2 = b2_scr[...]
    for j in range(_BB):
        xall = x_ref[:, j].reshape(_T * _D, _N).astype(jnp.bfloat16)
        yall = jax.lax.dot_general(xall, s16_scr[...], _RHS_T,
                                   preferred_element_type=jnp.float32)
        yall = yall.astype(jnp.bfloat16)
        for i in range(_T):
            lo = i * _D
            hi = min(lo + _P * _D, _T * _D)
            k = hi - lo
            ywin = yall[lo:hi]
            xwin = xall[lo:hi]
            lin = jnp.dot(w1[:, :k], ywin, preferred_element_type=jnp.float32) + b1
            inp2 = jnp.dot(w2[:, :k], xwin, preferred_element_type=jnp.float32) + b2
            xh = lin[:_O]
            gate = lin[_O:].astype(jnp.bfloat16)
            sg = jax.nn.sigmoid(gate).astype(jnp.float32)
            out_ref[i, j] = (xh + inp2) * sg


def kernel(x, supports, W1, b1, W2, b2):
    t, b, n, d = x.shape
    xp = jnp.transpose(x, (0, 1, 3, 2))
    b1r = b1.reshape(1, -1)
    b2r = b2.reshape(1, -1)

    out = pl.pallas_call(
        _gg_kernel,
        grid=(b // _BB,),
        in_specs=[
            pl.BlockSpec((t, _BB, d, n), lambda i: (0, i, 0, 0)),
            pl.BlockSpec((n, n), lambda i: (0, 0)),
            pl.BlockSpec((2 * _O, _P * d), lambda i: (0, 0)),
            pl.BlockSpec((1, 2 * _O), lambda i: (0, 0)),
            pl.BlockSpec((_O, _P * d), lambda i: (0, 0)),
            pl.BlockSpec((1, _O), lambda i: (0, 0)),
        ],
        out_specs=pl.BlockSpec((t, _BB, _O, n), lambda i: (0, i, 0, 0)),
        out_shape=jax.ShapeDtypeStruct((t, b, _O, n), jnp.float32),
        scratch_shapes=[
            pltpu.VMEM((n, n), jnp.bfloat16),
            pltpu.VMEM((2 * _O, _P * d), jnp.bfloat16),
            pltpu.VMEM((_O, _P * d), jnp.bfloat16),
            pltpu.VMEM((2 * _O, 1), jnp.float32),
            pltpu.VMEM((_O, 1), jnp.float32),
        ],
    )(xp, supports, W1, b1r, W2, b2r)
    return jnp.transpose(out, (0, 1, 3, 2))


# speedup vs baseline: 1.4386x; 1.0135x over previous
import jax
import jax.numpy as jnp
from jax.experimental import pallas as pl
from jax.experimental.pallas import tpu as pltpu

_T = 12
_B = 8
_N = 512
_D = 64
_P = 3
_O = 64

_RHS_T = (((1,), (1,)), ((), ()))
_BB = 2


def _gg_kernel(x_ref, s_ref, w1_ref, b1_ref, w2_ref, b2_ref, out_ref,
               s16_scr, w1_scr, w2_scr, b1_scr, b2_scr):
    @pl.when(pl.program_id(0) == 0)
    def _init():
        s16_scr[...] = s_ref[...].astype(jnp.bfloat16)
        w1_scr[...] = w1_ref[...].astype(jnp.bfloat16)
        w2_scr[...] = w2_ref[...].astype(jnp.bfloat16)
        b1_scr[...] = jnp.transpose(b1_ref[...], (1, 0))
        b2_scr[...] = jnp.transpose(b2_ref[...], (1, 0))

    w1 = w1_scr[...]
    w2 = w2_scr[...]
    b1 = b1_scr[...]
    b2 = b2_scr[...]
    for j in range(_BB):
        xall = x_ref[:, j].reshape(_T * _D, _N).astype(jnp.bfloat16)
        yall = jax.lax.dot_general(xall, s16_scr[...], _RHS_T,
                                   preferred_element_type=jnp.float32)
        yall = yall.astype(jnp.bfloat16)
        for i in range(_T):
            lo = i * _D
            hi = min(lo + _P * _D, _T * _D)
            k = hi - lo
            ywin = yall[lo:hi]
            xwin = xall[lo:hi]
            lin = jnp.dot(w1[:, :k], ywin, preferred_element_type=jnp.float32) + b1
            inp2 = jnp.dot(w2[:, :k], xwin, preferred_element_type=jnp.float32) + b2
            xh = lin[:_O]
            gate = lin[_O:]
            out_ref[i, j] = (xh + inp2) * jax.nn.sigmoid(gate)


def kernel(x, supports, W1, b1, W2, b2):
    t, b, n, d = x.shape
    xp = jnp.transpose(x, (0, 1, 3, 2))
    b1r = b1.reshape(1, -1)
    b2r = b2.reshape(1, -1)

    out = pl.pallas_call(
        _gg_kernel,
        grid=(b // _BB,),
        in_specs=[
            pl.BlockSpec((t, _BB, d, n), lambda i: (0, i, 0, 0)),
            pl.BlockSpec((n, n), lambda i: (0, 0)),
            pl.BlockSpec((2 * _O, _P * d), lambda i: (0, 0)),
            pl.BlockSpec((1, 2 * _O), lambda i: (0, 0)),
            pl.BlockSpec((_O, _P * d), lambda i: (0, 0)),
            pl.BlockSpec((1, _O), lambda i: (0, 0)),
        ],
        out_specs=pl.BlockSpec((t, _BB, _O, n), lambda i: (0, i, 0, 0)),
        out_shape=jax.ShapeDtypeStruct((t, b, _O, n), jnp.float32),
        scratch_shapes=[
            pltpu.VMEM((n, n), jnp.bfloat16),
            pltpu.VMEM((2 * _O, _P * d), jnp.bfloat16),
            pltpu.VMEM((_O, _P * d), jnp.bfloat16),
            pltpu.VMEM((2 * _O, 1), jnp.float32),
            pltpu.VMEM((_O, 1), jnp.float32),
        ],
    )(xp, supports, W1, b1r, W2, b2r)
    return jnp.transpose(out, (0, 1, 3, 2))
